# baseline (speedup 1.0000x reference)
import functools

import jax
import jax.numpy as jnp
from jax import lax
from jax.experimental import pallas as pl
from jax.experimental.pallas import tpu as pltpu
from jax.experimental.pallas import tpu_sc as plsc

EDGE_DIM = 64
NUM_WORKERS = 32
LANES = 16
COL_CHUNK = 512


def _bounds(seq_len):
    b0 = 2 * (seq_len - 1)
    b1 = b0 + 2 * (seq_len - 2)
    b2 = b1 + 2 * (seq_len - 3)
    return b0, b1, b2, b2 + seq_len


@functools.lru_cache(maxsize=None)
def _build_attr_call(seq_len):
    b0, b1, b2, num_edges = _bounds(seq_len)
    n_active = -(-num_edges // COL_CHUNK)
    e_pad = n_active * COL_CHUNK
    assert n_active <= NUM_WORKERS
    assert e_pad - num_edges < COL_CHUNK

    mesh = plsc.VectorSubcoreMesh(core_axis_name="c", subcore_axis_name="s")

    @functools.partial(
        pl.kernel,
        mesh=mesh,
        out_type=jax.ShapeDtypeStruct((EDGE_DIM, e_pad), jnp.float32),
        scratch_types=[
            pltpu.VMEM((5 * EDGE_DIM + LANES,), jnp.float32),
            pltpu.VMEM((EDGE_DIM, COL_CHUNK), jnp.float32),
        ],
    )
    def sc_kernel(emb_hbm, attr_hbm, emb_sm, cols_v):
        wid = lax.axis_index("s") * 2 + lax.axis_index("c")

        def type_of(c):
            return jnp.where(
                c < b0, 0, jnp.where(c < b1, 1, jnp.where(c < b2, 2, 0))
            )

        @pl.when(wid < n_active)
        def _():
            pltpu.sync_copy(emb_hbm, emb_sm.at[pl.ds(0, 5 * EDGE_DIM)])
            col_base = wid * COL_CHUNK

            def group_body(g, carry):
                cbase = col_base + g * LANES
                t_first = type_of(cbase)
                t_last = type_of(cbase + LANES - 1)

                @pl.when(t_first == t_last)
                def _():
                    base = t_first * EDGE_DIM
                    for d in range(EDGE_DIM):
                        v = emb_sm[pl.ds(base + d, LANES)]
                        cols_v[d, pl.ds(g * LANES, LANES)] = (
                            jnp.broadcast_to(v[0], (LANES,))
                        )

                @pl.when(t_first != t_last)
                def _():
                    tv = type_of(cbase + lax.iota(jnp.int32, LANES))
                    for d in range(EDGE_DIM):
                        s0 = emb_sm[pl.ds(d, LANES)][0]
                        s1 = emb_sm[pl.ds(EDGE_DIM + d, LANES)][0]
                        s2 = emb_sm[pl.ds(2 * EDGE_DIM + d, LANES)][0]
                        cols_v[d, pl.ds(g * LANES, LANES)] = jnp.where(
                            tv == 0, s0, jnp.where(tv == 1, s1, s2)
                        )

                return carry

            lax.fori_loop(0, COL_CHUNK // LANES, group_body, 0)

            pltpu.sync_copy(
                cols_v, attr_hbm.at[:, pl.ds(col_base, COL_CHUNK)]
            )

    return sc_kernel, e_pad


@functools.lru_cache(maxsize=None)
def _build_index_call(seq_len, e_pad):
    b0, b1, b2, num_edges = _bounds(seq_len)

    cb = 1024
    n_blocks = -(-num_edges // cb)
    assert n_blocks * cb <= e_pad or e_pad >= num_edges

    def tc_kernel(attr_in_ref, attr_out_ref, ei_ref):
        i = pl.program_id(0)
        attr_out_ref[...] = attr_in_ref[...]

        r = lax.broadcasted_iota(jnp.int32, (2, cb), 0)
        c = i * cb + lax.broadcasted_iota(jnp.int32, (2, cb), 1)
        in0 = c < b0
        in1 = c < b1
        in2 = c < b2
        s = jnp.where(in0, 0, jnp.where(in1, b0, jnp.where(in2, b1, b2)))
        w = jnp.where(in0, 1, jnp.where(in1, 2, jnp.where(in2, 3, 0)))
        local = c - s
        k = local >> 1
        p = local & 1
        ei_ref[...] = jnp.where(in2, k + (p ^ r) * w, local)

    return pl.pallas_call(
        tc_kernel,
        grid=(n_blocks,),
        in_specs=[
            pl.BlockSpec((EDGE_DIM, cb), lambda i: (0, i)),
        ],
        out_specs=[
            pl.BlockSpec((EDGE_DIM, cb), lambda i: (0, i)),
            pl.BlockSpec((2, cb), lambda i: (0, i)),
        ],
        out_shape=[
            jax.ShapeDtypeStruct((EDGE_DIM, num_edges), jnp.float32),
            jax.ShapeDtypeStruct((2, num_edges), jnp.int32),
        ],
    )


def kernel(token_ids, edge_emb):
    seq_len = token_ids.shape[1]
    attr_call, e_pad = _build_attr_call(seq_len)
    attr_t_pad = attr_call(edge_emb.reshape(-1))
    attr_t, edge_index = _build_index_call(seq_len, e_pad)(attr_t_pad)
    return (edge_index, attr_t.T)


# speedup vs baseline: 1.0990x; 1.0990x over previous
import functools

import jax
import jax.numpy as jnp
from jax import lax
from jax.experimental import pallas as pl
from jax.experimental.pallas import tpu as pltpu
from jax.experimental.pallas import tpu_sc as plsc

EDGE_DIM = 64
NUM_WORKERS = 32
LANES = 16
VPR = EDGE_DIM // LANES


def _round_up(x, m):
    return (x + m - 1) // m * m


def _bounds(seq_len):
    b0 = 2 * (seq_len - 1)
    b1 = b0 + 2 * (seq_len - 2)
    b2 = b1 + 2 * (seq_len - 3)
    return b0, b1, b2, b2 + seq_len


@functools.lru_cache(maxsize=None)
def _build_attr_call(seq_len):
    b0, b1, b2, num_edges = _bounds(seq_len)

    epw = _round_up(-(-num_edges // NUM_WORKERS), LANES)
    last_full = (num_edges - (NUM_WORKERS - 1) * epw) // 8 * 8
    assert 0 < last_full <= epw

    mesh = plsc.VectorSubcoreMesh(core_axis_name="c", subcore_axis_name="s")

    @functools.partial(
        pl.kernel,
        mesh=mesh,
        out_type=jax.ShapeDtypeStruct((num_edges, EDGE_DIM), jnp.float32),
        scratch_types=[
            pltpu.VMEM((5, EDGE_DIM), jnp.float32),
            pltpu.VMEM((epw, EDGE_DIM), jnp.float32),
        ],
    )
    def sc_kernel(emb_hbm, attr_hbm, emb_v, rows_v):
        wid = lax.axis_index("s") * 2 + lax.axis_index("c")
        edge_base = wid * epw

        pltpu.sync_copy(emb_hbm, emb_v)

        def type_of(c):
            return jnp.where(
                c < b0, 0, jnp.where(c < b1, 1, jnp.where(c < b2, 2, 0))
            )

        def group_body(g, carry):
            cbase = edge_base + g * LANES
            t_first = type_of(cbase)
            t_last = type_of(cbase + LANES - 1)

            @pl.when(t_first == t_last)
            def _():
                vs = [
                    emb_v[t_first, pl.ds(i * LANES, LANES)]
                    for i in range(VPR)
                ]
                for r in range(LANES):
                    for i in range(VPR):
                        rows_v[g * LANES + r, pl.ds(i * LANES, LANES)] = vs[i]

            @pl.when(t_first != t_last)
            def _():
                for r in range(LANES):
                    t_r = type_of(cbase + r)
                    for i in range(VPR):
                        rows_v[g * LANES + r, pl.ds(i * LANES, LANES)] = (
                            emb_v[t_r, pl.ds(i * LANES, LANES)]
                        )

            return carry

        lax.fori_loop(0, epw // LANES, group_body, 0)

        @pl.when(wid < NUM_WORKERS - 1)
        def _():
            pltpu.sync_copy(rows_v, attr_hbm.at[pl.ds(edge_base, epw)])

        @pl.when(wid == NUM_WORKERS - 1)
        def _():
            pltpu.sync_copy(
                rows_v.at[pl.ds(0, last_full)],
                attr_hbm.at[pl.ds(edge_base, last_full)],
            )

    return sc_kernel


@functools.lru_cache(maxsize=None)
def _build_index_call(seq_len):
    b0, b1, b2, num_edges = _bounds(seq_len)

    epw = _round_up(-(-num_edges // NUM_WORKERS), LANES)
    last_full = (num_edges - (NUM_WORKERS - 1) * epw) // 8 * 8
    tail_start = (NUM_WORKERS - 1) * epw + last_full
    assert tail_start % 8 == 0 and 0 < num_edges - tail_start < 8

    cb = 1024
    n_blocks = -(-num_edges // cb)
    src_col = tail_start - 1 - (n_blocks - 1) * cb
    assert 0 <= src_col < cb and tail_start - 1 >= b2

    def tc_kernel(attr_in_ref, attr_out_ref, ei_ref):
        i = pl.program_id(0)
        d = lax.broadcasted_iota(jnp.int32, (EDGE_DIM, EDGE_DIM), 0)
        e = lax.broadcasted_iota(jnp.int32, (EDGE_DIM, EDGE_DIM), 1)
        ident = (d == e).astype(jnp.float32)
        xt = lax.dot_general(
            ident,
            attr_in_ref[...],
            (((1,), (1,)), ((), ())),
            preferred_element_type=jnp.float32,
        )
        cols = i * cb + lax.broadcasted_iota(jnp.int32, (EDGE_DIM, cb), 1)
        patch = jnp.broadcast_to(xt[:, src_col : src_col + 1], xt.shape)
        attr_out_ref[...] = jnp.where(cols >= tail_start, patch, xt)

        r = lax.broadcasted_iota(jnp.int32, (2, cb), 0)
        c = i * cb + lax.broadcasted_iota(jnp.int32, (2, cb), 1)
        in0 = c < b0
        in1 = c < b1
        in2 = c < b2
        s = jnp.where(in0, 0, jnp.where(in1, b0, jnp.where(in2, b1, b2)))
        w = jnp.where(in0, 1, jnp.where(in1, 2, jnp.where(in2, 3, 0)))
        local = c - s
        k = local >> 1
        p = local & 1
        ei_ref[...] = jnp.where(in2, k + (p ^ r) * w, local)

    return pl.pallas_call(
        tc_kernel,
        grid=(n_blocks,),
        in_specs=[
            pl.BlockSpec((cb, EDGE_DIM), lambda i: (i, 0)),
        ],
        out_specs=[
            pl.BlockSpec((EDGE_DIM, cb), lambda i: (0, i)),
            pl.BlockSpec((2, cb), lambda i: (0, i)),
        ],
        out_shape=[
            jax.ShapeDtypeStruct((EDGE_DIM, num_edges), jnp.float32),
            jax.ShapeDtypeStruct((2, num_edges), jnp.int32),
        ],
    )


def kernel(token_ids, edge_emb):
    seq_len = token_ids.shape[1]
    attr_sc = _build_attr_call(seq_len)(edge_emb)
    attr_t, edge_index = _build_index_call(seq_len)(attr_sc)
    return (edge_index, attr_t.T)


# speedup vs baseline: 1.2269x; 1.1164x over previous
import functools

import jax
import jax.numpy as jnp
from jax import lax
from jax.experimental import pallas as pl
from jax.experimental.pallas import tpu as pltpu
from jax.experimental.pallas import tpu_sc as plsc

EDGE_DIM = 64
NUM_WORKERS = 32
LANES = 16
VPR = EDGE_DIM // LANES


def _round_up(x, m):
    return (x + m - 1) // m * m


def _bounds(seq_len):
    b0 = 2 * (seq_len - 1)
    b1 = b0 + 2 * (seq_len - 2)
    b2 = b1 + 2 * (seq_len - 3)
    return b0, b1, b2, b2 + seq_len


@functools.lru_cache(maxsize=None)
def _build_attr_call(seq_len):
    b0, b1, b2, num_edges = _bounds(seq_len)

    epw = _round_up(-(-num_edges // NUM_WORKERS), LANES)
    last_full = (num_edges - (NUM_WORKERS - 1) * epw) // 8 * 8
    assert 0 < last_full <= epw

    mesh = plsc.VectorSubcoreMesh(core_axis_name="c", subcore_axis_name="s")

    @functools.partial(
        pl.kernel,
        mesh=mesh,
        out_type=jax.ShapeDtypeStruct((num_edges, EDGE_DIM), jnp.float32),
        scratch_types=[
            pltpu.VMEM((5, EDGE_DIM), jnp.float32),
            pltpu.VMEM((epw, EDGE_DIM), jnp.float32),
            pltpu.SemaphoreType.DMA,
        ],
    )
    def sc_kernel(emb_hbm, attr_hbm, emb_v, rows_v, sem):
        wid = lax.axis_index("s") * 2 + lax.axis_index("c")
        edge_base = wid * epw

        pltpu.sync_copy(emb_hbm, emb_v)

        def type_of(c):
            return jnp.where(
                c < b0, 0, jnp.where(c < b1, 1, jnp.where(c < b2, 2, 0))
            )

        def group_body(g, carry):
            cbase = edge_base + g * LANES
            t_first = type_of(cbase)
            t_last = type_of(cbase + LANES - 1)

            @pl.when(t_first == t_last)
            def _():
                vs = [
                    emb_v[t_first, pl.ds(i * LANES, LANES)]
                    for i in range(VPR)
                ]
                for r in range(LANES):
                    for i in range(VPR):
                        rows_v[g * LANES + r, pl.ds(i * LANES, LANES)] = vs[i]

            @pl.when(t_first != t_last)
            def _():
                for r in range(LANES):
                    t_r = type_of(cbase + r)
                    for i in range(VPR):
                        rows_v[g * LANES + r, pl.ds(i * LANES, LANES)] = (
                            emb_v[t_r, pl.ds(i * LANES, LANES)]
                        )

            return carry

        half = epw // 2
        lax.fori_loop(0, epw // LANES // 2, group_body, 0)
        first = pltpu.async_copy(
            rows_v.at[pl.ds(0, half)],
            attr_hbm.at[pl.ds(edge_base, half)],
            sem,
        )
        lax.fori_loop(epw // LANES // 2, epw // LANES, group_body, 0)
        first.wait()

        @pl.when(wid < NUM_WORKERS - 1)
        def _():
            pltpu.sync_copy(
                rows_v.at[pl.ds(half, epw - half)],
                attr_hbm.at[pl.ds(edge_base + half, epw - half)],
            )

        @pl.when(wid == NUM_WORKERS - 1)
        def _():
            pltpu.sync_copy(
                rows_v.at[pl.ds(half, last_full - half)],
                attr_hbm.at[pl.ds(edge_base + half, last_full - half)],
            )

    return sc_kernel


@functools.lru_cache(maxsize=None)
def _build_index_call(seq_len):
    b0, b1, b2, num_edges = _bounds(seq_len)

    epw = _round_up(-(-num_edges // NUM_WORKERS), LANES)
    last_full = (num_edges - (NUM_WORKERS - 1) * epw) // 8 * 8
    tail_start = (NUM_WORKERS - 1) * epw + last_full
    assert tail_start % 8 == 0 and 0 < num_edges - tail_start < 8
    src_block = tail_start // 8 - 2
    assert src_block * 8 >= b2

    def tc_kernel(attr_in_ref, attr_out_ref, ei_ref):
        r = lax.broadcasted_iota(jnp.int32, (2, num_edges), 0)
        c = lax.broadcasted_iota(jnp.int32, (2, num_edges), 1)
        in0 = c < b0
        in1 = c < b1
        in2 = c < b2
        s = jnp.where(in0, 0, jnp.where(in1, b0, jnp.where(in2, b1, b2)))
        w = jnp.where(in0, 1, jnp.where(in1, 2, jnp.where(in2, 3, 0)))
        local = c - s
        k = local >> 1
        p = local & 1
        ei_ref[...] = jnp.where(in2, k + (p ^ r) * w, local)
        attr_out_ref[...] = attr_in_ref[...]

    return pl.pallas_call(
        tc_kernel,
        grid=(1,),
        in_specs=[
            pl.BlockSpec((8, EDGE_DIM), lambda i: (src_block, 0)),
        ],
        out_specs=[
            pl.BlockSpec((8, EDGE_DIM), lambda i: (tail_start // 8, 0)),
            pl.BlockSpec((2, num_edges), lambda i: (0, 0)),
        ],
        out_shape=[
            jax.ShapeDtypeStruct((num_edges, EDGE_DIM), jnp.float32),
            jax.ShapeDtypeStruct((2, num_edges), jnp.int32),
        ],
        input_output_aliases={0: 0},
    )


def kernel(token_ids, edge_emb):
    seq_len = token_ids.shape[1]
    attr_sc = _build_attr_call(seq_len)(edge_emb)
    edge_attr, edge_index = _build_index_call(seq_len)(attr_sc)
    return (edge_index, edge_attr)
